# baseline (speedup 1.0000x reference)
import jax
import jax.numpy as jnp
from jax import lax
from jax.experimental import pallas as pl
from jax.experimental.pallas import tpu as pltpu
from jax.experimental.pallas import tpu_sc as plsc

_B, _H, _W = 128, 32, 32
_S = _H * _W
_EMB = 128
_CHUNK = 128
_NCHUNK = _S // _CHUNK
_NW = 32
_KSPLIT = 120
_NB = _B - _KSPLIT
_TB = 4
_CB = 4
_NH = 4
_HCH = _EMB // _NH
_NG = _CHUNK // 16


def _sc_body(wk_hbm, cnt_hbm, cargo_hbm, road_hbm, res_hbm, mask_hbm, mtb_hbm,
             out_hbm, mtb_v, wk_v, cnt_v, cargo_v, road_v, res_v, mask_v,
             f_v, out_v, in_sem, out_sem):
    wid = lax.axis_index("s") * 2 + lax.axis_index("c")
    pltpu.sync_copy(mtb_hbm, mtb_v)
    n_chunks = (_NB * _NCHUNK) // _NW

    def in_copies(slot, t):
        gc = wid * n_chunks + t
        b = _KSPLIT + gc // _NCHUNK
        q = (gc % _NCHUNK) * _CHUNK
        sl = pl.ds(q, _CHUNK)
        sem = in_sem.at[slot]
        return [
            pltpu.make_async_copy(wk_hbm.at[b, :, sl], wk_v.at[slot], sem),
            pltpu.make_async_copy(cnt_hbm.at[b, :, sl], cnt_v.at[slot], sem),
            pltpu.make_async_copy(cargo_hbm.at[b, :, sl], cargo_v.at[slot],
                                  sem),
            pltpu.make_async_copy(road_hbm.at[b, :, sl], road_v.at[slot],
                                  sem),
            pltpu.make_async_copy(res_hbm.at[b, :, sl], res_v.at[slot], sem),
            pltpu.make_async_copy(mask_hbm.at[b, :, sl], mask_v.at[slot],
                                  sem),
        ]

    def out_copy(h, t):
        gc = wid * n_chunks + t
        b = gc // _NCHUNK
        q = (gc % _NCHUNK) * _CHUNK
        sl = pl.ds(q, _CHUNK)
        return pltpu.make_async_copy(
            out_v.at[h % 2], out_hbm.at[b, pl.ds(h * _HCH, _HCH), sl],
            out_sem.at[h % 2])

    for c in in_copies(0, 0):
        c.start()

    def chunk_body(t, _):
        slot = lax.rem(t, 2)

        @pl.when(t + 1 < n_chunks)
        def _():
            for c in in_copies(1 - slot, t + 1):
                c.start()

        for c in in_copies(slot, t):
            c.wait()

        for g in range(_NG):
            gs = pl.ds(g * 16, 16)
            msk = mask_v[slot, 0, gs]
            f_v[0, gs] = road_v[slot, 0, gs] * msk
            f_v[1, gs] = res_v[slot, 0, gs] * msk
            f_v[2, gs] = res_v[slot, 1, gs] * msk
            f_v[3, gs] = res_v[slot, 2, gs] * msk
            f_v[4, gs] = (wk_v[slot, 0, gs].astype(jnp.float32) *
                          cnt_v[slot, 0, gs] * msk)
            f_v[5, gs] = (wk_v[slot, 1, gs].astype(jnp.float32) *
                          cnt_v[slot, 1, gs] * msk)
            f_v[6, gs] = cargo_v[slot, 0, gs].astype(jnp.float32) * msk

        for h in range(_NH):
            fidx = 4 + h // (_NH // 2)

            @pl.when((t > 0) | (h >= 2))
            def _(h=h):
                tp = t if h >= 2 else t - 1
                hp = h - 2 if h >= 2 else h + 2
                out_copy(hp, tp).wait()

            def chan_body(co, _, h=h, fidx=fidx):
                c = co * _CB
                w = [[mtb_v[i, h * _HCH + c + k] for i in range(6)]
                     for k in range(_CB)]
                bias = [mtb_v[6, h * _HCH + c + k] for k in range(_CB)]

                for g in range(_NG):
                    gs = pl.ds(g * 16, 16)
                    f0 = f_v[0, gs]
                    f1 = f_v[1, gs]
                    f2 = f_v[2, gs]
                    f3 = f_v[3, gs]
                    f4 = f_v[fidx, gs]
                    f5 = f_v[6, gs]
                    for k in range(_CB):
                        out_v[h % 2, c + k, gs] = (
                            bias[k] + f0 * w[k][0] + f1 * w[k][1] +
                            f2 * w[k][2] + f3 * w[k][3] + f4 * w[k][4] +
                            f5 * w[k][5])
                return 0

            lax.fori_loop(0, _HCH // _CB, chan_body, 0)
            out_copy(h, t).start()
        return 0

    lax.fori_loop(0, n_chunks, chunk_body, 0)
    for h in range(2, _NH):
        out_copy(h, n_chunks - 1).wait()


def _tc_body(wk_ref, cnt_ref, cargo_ref, road_ref, res_ref, mask_ref, mt_ref,
             out_ref):
    mt = mt_ref[...]
    for tb in range(_TB):
        mask = mask_ref[tb]
        wkb = wk_ref[tb].astype(jnp.float32)
        cnt = cnt_ref[tb]
        res = res_ref[tb]
        f0 = road_ref[tb] * mask
        f1 = res[0:1] * mask
        f2 = res[1:2] * mask
        f3 = res[2:3] * mask
        f4 = wkb[0:1] * cnt[0:1] * mask
        f5 = wkb[1:2] * cnt[1:2] * mask
        f6 = cargo_ref[tb].astype(jnp.float32) * mask
        acc = jnp.broadcast_to(mt[:, 7:8], (_EMB, _S))
        for i, f in enumerate((f0, f1, f2, f3, f4, f5, f6)):
            acc = acc + mt[:, i:i + 1] * f
        out_ref[tb] = acc


def _tc_call(wk, cnt, cargo, road, res, mask, mt):
    grid = (_KSPLIT // _TB,)
    bs = lambda k: pl.BlockSpec((_TB, k, _S), lambda i: (i, 0, 0))
    return pl.pallas_call(
        _tc_body,
        grid=grid,
        in_specs=[
            bs(2), bs(2), bs(1), bs(1), bs(3), bs(1),
            pl.BlockSpec((_EMB, 8), lambda i: (0, 0)),
        ],
        out_specs=pl.BlockSpec((_TB, _EMB, _S), lambda i: (i, 0, 0)),
        out_shape=jax.ShapeDtypeStruct((_KSPLIT, _EMB, _S), jnp.float32),
        compiler_params=pltpu.CompilerParams(
            dimension_semantics=("parallel",)),
    )(wk, cnt, cargo, road, res, mask, mt)


def _sc_call(wk, cnt, cargo, road, res, mask, mtb):
    mesh = plsc.VectorSubcoreMesh(core_axis_name="c", subcore_axis_name="s")
    fn = pl.kernel(
        _sc_body,
        out_type=jax.ShapeDtypeStruct((_NB, _EMB, _S), jnp.float32),
        mesh=mesh,
        scratch_types=[
            pltpu.VMEM((7, _EMB, 16), jnp.float32),
            pltpu.VMEM((2, 2, _CHUNK), jnp.int32),
            pltpu.VMEM((2, 2, _CHUNK), jnp.float32),
            pltpu.VMEM((2, 1, _CHUNK), jnp.int32),
            pltpu.VMEM((2, 1, _CHUNK), jnp.float32),
            pltpu.VMEM((2, 3, _CHUNK), jnp.float32),
            pltpu.VMEM((2, 1, _CHUNK), jnp.float32),
            pltpu.VMEM((7, _CHUNK), jnp.float32),
            pltpu.VMEM((2, _HCH, _CHUNK), jnp.float32),
            pltpu.SemaphoreType.DMA((2,)),
            pltpu.SemaphoreType.DMA((2,)),
        ],
    )
    return fn(wk, cnt, cargo, road, res, mask, mtb)


def kernel(worker, worker_COUNT, worker_cargo_full, road_level, resources,
           input_mask, emb_worker, emb_cargo, conv_w, conv_b):
    wk = worker.reshape(_B, 2, _S)
    cnt = worker_COUNT.reshape(_B, 2, _S)
    cargo = worker_cargo_full.reshape(_B, 1, _S)
    road = road_level.reshape(_B, 1, _S)
    res = resources.reshape(_B, 3, _S)
    mask = input_mask.reshape(_B, 1, _S)

    ew1 = emb_worker[1]
    ec1 = emb_cargo[1]
    wsel = jnp.concatenate([ew1, ew1])
    mt7 = jnp.concatenate(
        [conv_w, wsel[:, None], ec1[:, None], conv_b[:, None]],
        axis=1)
    mtb = jnp.broadcast_to(mt7.T[:, :, None], (7, _EMB, 16))

    zeros = jnp.zeros((_EMB // 2,), jnp.float32)
    col4 = jnp.concatenate([ew1, zeros])
    col5 = jnp.concatenate([zeros, ew1])
    mt8 = jnp.concatenate(
        [conv_w, col4[:, None], col5[:, None], ec1[:, None], conv_b[:, None]],
        axis=1)

    sc_out = _sc_call(wk, cnt, cargo, road, res, mask, mtb)
    tc_out = _tc_call(wk, cnt, cargo, road, res, mask, mt8)
    out = jnp.concatenate([tc_out, sc_out], axis=0)
    return out.reshape(_B, _EMB, _H, _W), input_mask


# speedup vs baseline: 1.0055x; 1.0055x over previous
import jax
import jax.numpy as jnp
from jax import lax
from jax.experimental import pallas as pl
from jax.experimental.pallas import tpu as pltpu
from jax.experimental.pallas import tpu_sc as plsc

_B, _H, _W = 128, 32, 32
_S = _H * _W
_EMB = 128
_CHUNK = 128
_NCHUNK = _S // _CHUNK
_NW = 32
_KSPLIT = 104
_NB = _B - _KSPLIT
_TB = 4
_CB = 4
_NH = 4
_HCH = _EMB // _NH
_NG = _CHUNK // 16


def _sc_body(wk_hbm, cnt_hbm, cargo_hbm, road_hbm, res_hbm, mask_hbm, mtb_hbm,
             out_hbm, mtb_v, wk_v, cnt_v, cargo_v, road_v, res_v, mask_v,
             f_v, out_v, in_sem, out_sem):
    wid = lax.axis_index("s") * 2 + lax.axis_index("c")
    pltpu.sync_copy(mtb_hbm, mtb_v)
    n_chunks = (_NB * _NCHUNK) // _NW

    def in_copies(slot, t):
        gc = wid * n_chunks + t
        b = _KSPLIT + gc // _NCHUNK
        q = (gc % _NCHUNK) * _CHUNK
        sl = pl.ds(q, _CHUNK)
        sem = in_sem.at[slot]
        return [
            pltpu.make_async_copy(wk_hbm.at[b, :, sl], wk_v.at[slot], sem),
            pltpu.make_async_copy(cnt_hbm.at[b, :, sl], cnt_v.at[slot], sem),
            pltpu.make_async_copy(cargo_hbm.at[b, :, sl], cargo_v.at[slot],
                                  sem),
            pltpu.make_async_copy(road_hbm.at[b, :, sl], road_v.at[slot],
                                  sem),
            pltpu.make_async_copy(res_hbm.at[b, :, sl], res_v.at[slot], sem),
            pltpu.make_async_copy(mask_hbm.at[b, :, sl], mask_v.at[slot],
                                  sem),
        ]

    def out_copy(h, t):
        gc = wid * n_chunks + t
        b = gc // _NCHUNK
        q = (gc % _NCHUNK) * _CHUNK
        sl = pl.ds(q, _CHUNK)
        return pltpu.make_async_copy(
            out_v.at[h % 2], out_hbm.at[b, pl.ds(h * _HCH, _HCH), sl],
            out_sem.at[h % 2])

    for c in in_copies(0, 0):
        c.start()

    def chunk_body(t, _):
        slot = lax.rem(t, 2)

        @pl.when(t + 1 < n_chunks)
        def _():
            for c in in_copies(1 - slot, t + 1):
                c.start()

        for c in in_copies(slot, t):
            c.wait()

        for g in range(_NG):
            gs = pl.ds(g * 16, 16)
            msk = mask_v[slot, 0, gs]
            f_v[0, gs] = road_v[slot, 0, gs] * msk
            f_v[1, gs] = res_v[slot, 0, gs] * msk
            f_v[2, gs] = res_v[slot, 1, gs] * msk
            f_v[3, gs] = res_v[slot, 2, gs] * msk
            f_v[4, gs] = (wk_v[slot, 0, gs].astype(jnp.float32) *
                          cnt_v[slot, 0, gs] * msk)
            f_v[5, gs] = (wk_v[slot, 1, gs].astype(jnp.float32) *
                          cnt_v[slot, 1, gs] * msk)
            f_v[6, gs] = cargo_v[slot, 0, gs].astype(jnp.float32) * msk

        for h in range(_NH):
            fidx = 4 + h // (_NH // 2)

            @pl.when((t > 0) | (h >= 2))
            def _(h=h):
                tp = t if h >= 2 else t - 1
                hp = h - 2 if h >= 2 else h + 2
                out_copy(hp, tp).wait()

            def chan_body(co, _, h=h, fidx=fidx):
                c = co * _CB
                w = [[mtb_v[i, h * _HCH + c + k] for i in range(6)]
                     for k in range(_CB)]
                bias = [mtb_v[6, h * _HCH + c + k] for k in range(_CB)]

                for g in range(_NG):
                    gs = pl.ds(g * 16, 16)
                    f0 = f_v[0, gs]
                    f1 = f_v[1, gs]
                    f2 = f_v[2, gs]
                    f3 = f_v[3, gs]
                    f4 = f_v[fidx, gs]
                    f5 = f_v[6, gs]
                    for k in range(_CB):
                        out_v[h % 2, c + k, gs] = (
                            bias[k] + f0 * w[k][0] + f1 * w[k][1] +
                            f2 * w[k][2] + f3 * w[k][3] + f4 * w[k][4] +
                            f5 * w[k][5])
                return 0

            lax.fori_loop(0, _HCH // _CB, chan_body, 0)
            out_copy(h, t).start()
        return 0

    lax.fori_loop(0, n_chunks, chunk_body, 0)
    for h in range(2, _NH):
        out_copy(h, n_chunks - 1).wait()


def _tc_body(wk_ref, cnt_ref, cargo_ref, road_ref, res_ref, mask_ref, mt_ref,
             out_ref):
    mt = mt_ref[...]
    for tb in range(_TB):
        mask = mask_ref[tb]
        wkb = wk_ref[tb].astype(jnp.float32)
        cnt = cnt_ref[tb]
        res = res_ref[tb]
        f0 = road_ref[tb] * mask
        f1 = res[0:1] * mask
        f2 = res[1:2] * mask
        f3 = res[2:3] * mask
        f4 = wkb[0:1] * cnt[0:1] * mask
        f5 = wkb[1:2] * cnt[1:2] * mask
        f6 = cargo_ref[tb].astype(jnp.float32) * mask
        acc = jnp.broadcast_to(mt[:, 7:8], (_EMB, _S))
        for i, f in enumerate((f0, f1, f2, f3, f4, f5, f6)):
            acc = acc + mt[:, i:i + 1] * f
        out_ref[tb] = acc


def _tc_call(wk, cnt, cargo, road, res, mask, mt):
    grid = (_KSPLIT // _TB,)
    bs = lambda k: pl.BlockSpec((_TB, k, _S), lambda i: (i, 0, 0))
    return pl.pallas_call(
        _tc_body,
        grid=grid,
        in_specs=[
            bs(2), bs(2), bs(1), bs(1), bs(3), bs(1),
            pl.BlockSpec((_EMB, 8), lambda i: (0, 0)),
        ],
        out_specs=pl.BlockSpec((_TB, _EMB, _S), lambda i: (i, 0, 0)),
        out_shape=jax.ShapeDtypeStruct((_KSPLIT, _EMB, _S), jnp.float32),
        compiler_params=pltpu.CompilerParams(
            dimension_semantics=("parallel",)),
    )(wk, cnt, cargo, road, res, mask, mt)


def _sc_call(wk, cnt, cargo, road, res, mask, mtb):
    mesh = plsc.VectorSubcoreMesh(core_axis_name="c", subcore_axis_name="s")
    fn = pl.kernel(
        _sc_body,
        out_type=jax.ShapeDtypeStruct((_NB, _EMB, _S), jnp.float32),
        mesh=mesh,
        scratch_types=[
            pltpu.VMEM((7, _EMB, 16), jnp.float32),
            pltpu.VMEM((2, 2, _CHUNK), jnp.int32),
            pltpu.VMEM((2, 2, _CHUNK), jnp.float32),
            pltpu.VMEM((2, 1, _CHUNK), jnp.int32),
            pltpu.VMEM((2, 1, _CHUNK), jnp.float32),
            pltpu.VMEM((2, 3, _CHUNK), jnp.float32),
            pltpu.VMEM((2, 1, _CHUNK), jnp.float32),
            pltpu.VMEM((7, _CHUNK), jnp.float32),
            pltpu.VMEM((2, _HCH, _CHUNK), jnp.float32),
            pltpu.SemaphoreType.DMA((2,)),
            pltpu.SemaphoreType.DMA((2,)),
        ],
    )
    return fn(wk, cnt, cargo, road, res, mask, mtb)


def kernel(worker, worker_COUNT, worker_cargo_full, road_level, resources,
           input_mask, emb_worker, emb_cargo, conv_w, conv_b):
    wk = worker.reshape(_B, 2, _S)
    cnt = worker_COUNT.reshape(_B, 2, _S)
    cargo = worker_cargo_full.reshape(_B, 1, _S)
    road = road_level.reshape(_B, 1, _S)
    res = resources.reshape(_B, 3, _S)
    mask = input_mask.reshape(_B, 1, _S)

    ew1 = emb_worker[1]
    ec1 = emb_cargo[1]
    wsel = jnp.concatenate([ew1, ew1])
    mt7 = jnp.concatenate(
        [conv_w, wsel[:, None], ec1[:, None], conv_b[:, None]],
        axis=1)
    mtb = jnp.broadcast_to(mt7.T[:, :, None], (7, _EMB, 16))

    zeros = jnp.zeros((_EMB // 2,), jnp.float32)
    col4 = jnp.concatenate([ew1, zeros])
    col5 = jnp.concatenate([zeros, ew1])
    mt8 = jnp.concatenate(
        [conv_w, col4[:, None], col5[:, None], ec1[:, None], conv_b[:, None]],
        axis=1)

    sc_out = _sc_call(wk, cnt, cargo, road, res, mask, mtb)
    tc_out = _tc_call(wk, cnt, cargo, road, res, mask, mt8)
    out = jnp.concatenate([tc_out, sc_out], axis=0)
    return out.reshape(_B, _EMB, _H, _W), input_mask


# speedup vs baseline: 1.0150x; 1.0095x over previous
import jax
import jax.numpy as jnp
from jax import lax
from jax.experimental import pallas as pl
from jax.experimental.pallas import tpu as pltpu
from jax.experimental.pallas import tpu_sc as plsc

_B, _H, _W = 128, 32, 32
_S = _H * _W
_EMB = 128
_CHUNK = 128
_NCHUNK = _S // _CHUNK
_NW = 32
_KSPLIT = 112
_NB = _B - _KSPLIT
_TB = 4
_CB = 4
_NH = 4
_HCH = _EMB // _NH
_NG = _CHUNK // 16


def _sc_body(wk_hbm, cnt_hbm, cargo_hbm, road_hbm, res_hbm, mask_hbm, mtb_hbm,
             out_hbm, mtb_v, wk_v, cnt_v, cargo_v, road_v, res_v, mask_v,
             f_v, out_v, in_sem, out_sem):
    wid = lax.axis_index("s") * 2 + lax.axis_index("c")
    pltpu.sync_copy(mtb_hbm, mtb_v)
    n_chunks = (_NB * _NCHUNK) // _NW

    def in_copies(slot, t):
        gc = wid * n_chunks + t
        b = _KSPLIT + gc // _NCHUNK
        q = (gc % _NCHUNK) * _CHUNK
        sl = pl.ds(q, _CHUNK)
        sem = in_sem.at[slot]
        return [
            pltpu.make_async_copy(wk_hbm.at[b, :, sl], wk_v.at[slot], sem),
            pltpu.make_async_copy(cnt_hbm.at[b, :, sl], cnt_v.at[slot], sem),
            pltpu.make_async_copy(cargo_hbm.at[b, :, sl], cargo_v.at[slot],
                                  sem),
            pltpu.make_async_copy(road_hbm.at[b, :, sl], road_v.at[slot],
                                  sem),
            pltpu.make_async_copy(res_hbm.at[b, :, sl], res_v.at[slot], sem),
            pltpu.make_async_copy(mask_hbm.at[b, :, sl], mask_v.at[slot],
                                  sem),
        ]

    def out_copy(h, t):
        gc = wid * n_chunks + t
        b = gc // _NCHUNK
        q = (gc % _NCHUNK) * _CHUNK
        sl = pl.ds(q, _CHUNK)
        return pltpu.make_async_copy(
            out_v.at[h % 2], out_hbm.at[b, pl.ds(h * _HCH, _HCH), sl],
            out_sem.at[h % 2])

    for c in in_copies(0, 0):
        c.start()

    def chunk_body(t, _):
        slot = lax.rem(t, 2)

        @pl.when(t + 1 < n_chunks)
        def _():
            for c in in_copies(1 - slot, t + 1):
                c.start()

        for c in in_copies(slot, t):
            c.wait()

        for g in range(_NG):
            gs = pl.ds(g * 16, 16)
            msk = mask_v[slot, 0, gs]
            f_v[0, gs] = road_v[slot, 0, gs] * msk
            f_v[1, gs] = res_v[slot, 0, gs] * msk
            f_v[2, gs] = res_v[slot, 1, gs] * msk
            f_v[3, gs] = res_v[slot, 2, gs] * msk
            f_v[4, gs] = (wk_v[slot, 0, gs].astype(jnp.float32) *
                          cnt_v[slot, 0, gs] * msk)
            f_v[5, gs] = (wk_v[slot, 1, gs].astype(jnp.float32) *
                          cnt_v[slot, 1, gs] * msk)
            f_v[6, gs] = cargo_v[slot, 0, gs].astype(jnp.float32) * msk

        for h in range(_NH):
            fidx = 4 + h // (_NH // 2)

            @pl.when((t > 0) | (h >= 2))
            def _(h=h):
                tp = t if h >= 2 else t - 1
                hp = h - 2 if h >= 2 else h + 2
                out_copy(hp, tp).wait()

            def chan_body(co, _, h=h, fidx=fidx):
                c = co * _CB
                w = [[mtb_v[i, h * _HCH + c + k] for i in range(6)]
                     for k in range(_CB)]
                bias = [mtb_v[6, h * _HCH + c + k] for k in range(_CB)]

                for g in range(_NG):
                    gs = pl.ds(g * 16, 16)
                    f0 = f_v[0, gs]
                    f1 = f_v[1, gs]
                    f2 = f_v[2, gs]
                    f3 = f_v[3, gs]
                    f4 = f_v[fidx, gs]
                    f5 = f_v[6, gs]
                    for k in range(_CB):
                        out_v[h % 2, c + k, gs] = (
                            bias[k] + f0 * w[k][0] + f1 * w[k][1] +
                            f2 * w[k][2] + f3 * w[k][3] + f4 * w[k][4] +
                            f5 * w[k][5])
                return 0

            lax.fori_loop(0, _HCH // _CB, chan_body, 0)
            out_copy(h, t).start()
        return 0

    lax.fori_loop(0, n_chunks, chunk_body, 0)
    for h in range(2, _NH):
        out_copy(h, n_chunks - 1).wait()


def _tc_body(wk_ref, cnt_ref, cargo_ref, road_ref, res_ref, mask_ref, mt_ref,
             out_ref):
    mt = mt_ref[...]
    for tb in range(_TB):
        mask = mask_ref[tb]
        wkb = wk_ref[tb].astype(jnp.float32)
        cnt = cnt_ref[tb]
        res = res_ref[tb]
        f0 = road_ref[tb] * mask
        f1 = res[0:1] * mask
        f2 = res[1:2] * mask
        f3 = res[2:3] * mask
        f4 = wkb[0:1] * cnt[0:1] * mask
        f5 = wkb[1:2] * cnt[1:2] * mask
        f6 = cargo_ref[tb].astype(jnp.float32) * mask
        acc = jnp.broadcast_to(mt[:, 7:8], (_EMB, _S))
        for i, f in enumerate((f0, f1, f2, f3, f4, f5, f6)):
            acc = acc + mt[:, i:i + 1] * f
        out_ref[tb] = acc


def _tc_call(wk, cnt, cargo, road, res, mask, mt):
    grid = (_KSPLIT // _TB,)
    bs = lambda k: pl.BlockSpec((_TB, k, _S), lambda i: (i, 0, 0))
    return pl.pallas_call(
        _tc_body,
        grid=grid,
        in_specs=[
            bs(2), bs(2), bs(1), bs(1), bs(3), bs(1),
            pl.BlockSpec((_EMB, 8), lambda i: (0, 0)),
        ],
        out_specs=pl.BlockSpec((_TB, _EMB, _S), lambda i: (i, 0, 0)),
        out_shape=jax.ShapeDtypeStruct((_KSPLIT, _EMB, _S), jnp.float32),
        compiler_params=pltpu.CompilerParams(
            dimension_semantics=("parallel",)),
    )(wk, cnt, cargo, road, res, mask, mt)


def _sc_call(wk, cnt, cargo, road, res, mask, mtb):
    mesh = plsc.VectorSubcoreMesh(core_axis_name="c", subcore_axis_name="s")
    fn = pl.kernel(
        _sc_body,
        out_type=jax.ShapeDtypeStruct((_NB, _EMB, _S), jnp.float32),
        mesh=mesh,
        scratch_types=[
            pltpu.VMEM((7, _EMB, 16), jnp.float32),
            pltpu.VMEM((2, 2, _CHUNK), jnp.int32),
            pltpu.VMEM((2, 2, _CHUNK), jnp.float32),
            pltpu.VMEM((2, 1, _CHUNK), jnp.int32),
            pltpu.VMEM((2, 1, _CHUNK), jnp.float32),
            pltpu.VMEM((2, 3, _CHUNK), jnp.float32),
            pltpu.VMEM((2, 1, _CHUNK), jnp.float32),
            pltpu.VMEM((7, _CHUNK), jnp.float32),
            pltpu.VMEM((2, _HCH, _CHUNK), jnp.float32),
            pltpu.SemaphoreType.DMA((2,)),
            pltpu.SemaphoreType.DMA((2,)),
        ],
    )
    return fn(wk, cnt, cargo, road, res, mask, mtb)


def kernel(worker, worker_COUNT, worker_cargo_full, road_level, resources,
           input_mask, emb_worker, emb_cargo, conv_w, conv_b):
    wk = worker.reshape(_B, 2, _S)
    cnt = worker_COUNT.reshape(_B, 2, _S)
    cargo = worker_cargo_full.reshape(_B, 1, _S)
    road = road_level.reshape(_B, 1, _S)
    res = resources.reshape(_B, 3, _S)
    mask = input_mask.reshape(_B, 1, _S)

    ew1 = emb_worker[1]
    ec1 = emb_cargo[1]
    wsel = jnp.concatenate([ew1, ew1])
    mt7 = jnp.concatenate(
        [conv_w, wsel[:, None], ec1[:, None], conv_b[:, None]],
        axis=1)
    mtb = jnp.broadcast_to(mt7.T[:, :, None], (7, _EMB, 16))

    zeros = jnp.zeros((_EMB // 2,), jnp.float32)
    col4 = jnp.concatenate([ew1, zeros])
    col5 = jnp.concatenate([zeros, ew1])
    mt8 = jnp.concatenate(
        [conv_w, col4[:, None], col5[:, None], ec1[:, None], conv_b[:, None]],
        axis=1)

    sc_out = _sc_call(wk, cnt, cargo, road, res, mask, mtb)
    tc_out = _tc_call(wk, cnt, cargo, road, res, mask, mt8)
    out = jnp.concatenate([tc_out, sc_out], axis=0)
    return out.reshape(_B, _EMB, _H, _W), input_mask
